# Initial kernel scaffold; baseline (speedup 1.0000x reference)
import jax
import jax.numpy as jnp
from jax.experimental import pallas as pl


def kernel(embed_t, h_t, context, W_x, W_h, W_ctx, b_ctx):
    raise NotImplementedError("write your pallas kernel here")


# speedup vs baseline: 1.1741x; 1.1741x over previous
import functools

import jax
import jax.numpy as jnp
from jax.experimental import pallas as pl
from jax.experimental.pallas import tpu as pltpu

B = 16384
EMBED = 512
HIDDEN = 1024
CTX = 1024

TILE = 1024


def _gate_body(e_ref, h_ref, c_ref, wx_ref, wh_ref, wc_ref, b_ref, o_ref):
    s = jnp.sum(e_ref[...] * wx_ref[...], axis=1)
    s = s + jnp.sum(h_ref[...] * wh_ref[...], axis=1)
    s = s + jnp.sum(c_ref[...] * wc_ref[...], axis=1)
    o_ref[...] = jax.nn.sigmoid(s + b_ref[0, 0])


@jax.jit
def _gate_tc(embed_t, h_t, context, W_x, W_h, W_ctx, b2):
    grid = (B // TILE,)
    return pl.pallas_call(
        _gate_body,
        grid=grid,
        in_specs=[
            pl.BlockSpec((TILE, EMBED), lambda i: (i, 0)),
            pl.BlockSpec((TILE, HIDDEN), lambda i: (i, 0)),
            pl.BlockSpec((TILE, CTX), lambda i: (i, 0)),
            pl.BlockSpec((1, EMBED), lambda i: (0, 0)),
            pl.BlockSpec((1, HIDDEN), lambda i: (0, 0)),
            pl.BlockSpec((1, CTX), lambda i: (0, 0)),
            pl.BlockSpec((1, 1), lambda i: (0, 0)),
        ],
        out_specs=pl.BlockSpec((TILE,), lambda i: (i,)),
        out_shape=jax.ShapeDtypeStruct((B,), jnp.float32),
        compiler_params=pltpu.CompilerParams(
            dimension_semantics=("arbitrary",),
        ),
    )(embed_t, h_t, context, W_x, W_h, W_ctx, b2)


def kernel(embed_t, h_t, context, W_x, W_h, W_ctx, b_ctx):
    b2 = b_ctx.reshape(1, 1)
    return _gate_tc(embed_t, h_t, context, W_x, W_h, W_ctx, b2)
